# baseline (speedup 1.0000x reference)
import functools

import jax
import jax.numpy as jnp
from jax import lax
from jax.experimental import pallas as pl
from jax.experimental.pallas import tpu as pltpu
from jax.experimental.pallas import tpu_sc as plsc


def _gather_kernel(N, D, n_per_w, chunk, n_chunks):
    mesh = plsc.VectorSubcoreMesh(core_axis_name="c", subcore_axis_name="s")

    @functools.partial(
        pl.kernel,
        mesh=mesh,
        out_type=jax.ShapeDtypeStruct((N, D), jnp.float32),
        scratch_types=[
            pltpu.VMEM((chunk,), jnp.int32),
            pltpu.VMEM((chunk, D), jnp.float32),
            pltpu.SemaphoreType.DMA,
        ],
        compiler_params=pltpu.CompilerParams(use_tc_tiling_on_sc=False),
    )
    def k(idx_hbm, tab_hbm, out_hbm, idx_v, rows_v, sem):
        nc = lax.axis_size("c")
        wid = lax.axis_index("s") * nc + lax.axis_index("c")
        base = wid * n_per_w

        @pl.loop(0, n_chunks)
        def _(i):
            off = base + i * chunk
            pltpu.sync_copy(idx_hbm.at[pl.ds(off, chunk)], idx_v)
            pltpu.async_copy(tab_hbm.at[idx_v], rows_v, sem).wait()
            pltpu.sync_copy(rows_v, out_hbm.at[pl.ds(off, chunk)])

    return k


def kernel(token_ids, weights):
    B, S = token_ids.shape
    V, D = weights.shape
    N = B * S
    idx = token_ids.reshape(N).astype(jnp.int32)

    NW = 32
    n_per_w = N // NW
    chunk = 512
    n_chunks = n_per_w // chunk

    out = _gather_kernel(N, D, n_per_w, chunk, n_chunks)(idx, weights)
    return out.reshape(B, S, D)


# speedup vs baseline: 1.0332x; 1.0332x over previous
import functools

import jax
import jax.numpy as jnp
from jax import lax
from jax.experimental import pallas as pl
from jax.experimental.pallas import tpu as pltpu
from jax.experimental.pallas import tpu_sc as plsc

_NBUF = 2


def _gather_kernel(N, D, n_per_w, chunk, n_chunks):
    mesh = plsc.VectorSubcoreMesh(core_axis_name="c", subcore_axis_name="s")

    @functools.partial(
        pl.kernel,
        mesh=mesh,
        out_type=jax.ShapeDtypeStruct((N, D), jnp.float32),
        scratch_types=[
            [pltpu.VMEM((chunk,), jnp.int32) for _ in range(_NBUF)],
            [pltpu.VMEM((chunk, D), jnp.float32) for _ in range(_NBUF)],
            [pltpu.SemaphoreType.DMA for _ in range(_NBUF)],
        ],
        compiler_params=pltpu.CompilerParams(use_tc_tiling_on_sc=False),
    )
    def k(idx_hbm, tab_hbm, out_hbm, idx_bufs, rows, sems):
        nc = lax.axis_size("c")
        wid = lax.axis_index("s") * nc + lax.axis_index("c")
        base = wid * n_per_w

        def fire(j, b):
            pltpu.sync_copy(idx_hbm.at[pl.ds(base + j * chunk, chunk)], idx_bufs[b])
            pltpu.async_copy(tab_hbm.at[idx_bufs[b]], rows[b], sems[b])

        for b in range(_NBUF):
            fire(b, b)

        @pl.loop(0, n_chunks, step=_NBUF)
        def _(i):
            for b in range(_NBUF):
                j = i + b
                pltpu.make_async_copy(
                    tab_hbm.at[idx_bufs[b]], rows[b], sems[b]
                ).wait()
                pltpu.sync_copy(rows[b], out_hbm.at[pl.ds(base + j * chunk, chunk)])

                @pl.when(j + _NBUF < n_chunks)
                def _():
                    fire(j + _NBUF, b)

    return k


def kernel(token_ids, weights):
    B, S = token_ids.shape
    V, D = weights.shape
    N = B * S
    idx = token_ids.reshape(N).astype(jnp.int32)

    NW = 32
    n_per_w = N // NW
    chunk = 512
    n_chunks = n_per_w // chunk
    assert n_per_w % chunk == 0 and n_chunks % _NBUF == 0

    out = _gather_kernel(N, D, n_per_w, chunk, n_chunks)(idx, weights)
    return out.reshape(B, S, D)
